# baseline (speedup 1.0000x reference)
import functools

import jax
import jax.numpy as jnp
from jax import lax
from jax.experimental import pallas as pl
from jax.experimental.pallas import tpu as pltpu
from jax.experimental.pallas import tpu_sc as plsc

_NC = 2
_NS = 16
_NW = _NC * _NS
_DPAD = 1024
_SPAD = 56


@functools.partial(jax.jit, static_argnames=("nb",))
def _gather_rows(table, idx2d, nb):
    b_per_w = nb // _NW
    assert b_per_w % 2 == 0
    n_pairs = b_per_w // 2
    idx_len = idx2d.shape[1]
    mesh = plsc.VectorSubcoreMesh(
        core_axis_name="c", subcore_axis_name="s",
        num_cores=_NC, num_subcores=_NS)

    @functools.partial(
        pl.kernel,
        out_type=jax.ShapeDtypeStruct((nb, _SPAD, _DPAD), jnp.float32),
        mesh=mesh,
        scratch_types=[
            pltpu.VMEM((idx_len,), jnp.int32),
            pltpu.VMEM((2, _SPAD, _DPAD), jnp.float32),
            pltpu.SemaphoreType.DMA((2,)),
            pltpu.SemaphoreType.DMA((2,)),
        ],
        compiler_params=pltpu.CompilerParams(use_tc_tiling_on_sc=True),
    )
    def run(table_hbm, idx_hbm, out_hbm, idx_v, rows_v, gsem, ssem):
        wid = lax.axis_index("s") * _NC + lax.axis_index("c")
        base = wid * b_per_w
        pltpu.sync_copy(idx_hbm.at[wid], idx_v)

        def gather(buf, k):
            return pltpu.make_async_copy(
                table_hbm.at[idx_v.at[pl.ds(k * _SPAD, _SPAD)]],
                rows_v.at[buf], gsem.at[buf])

        def store(buf, k):
            return pltpu.make_async_copy(
                rows_v.at[buf], out_hbm.at[base + k], ssem.at[buf])

        gather(0, 0).start()
        gather(1, 1).start()

        @pl.loop(0, n_pairs)
        def _pair(g):
            k0 = 2 * g
            k1 = k0 + 1
            last = b_per_w - 1
            k2 = jnp.minimum(k0 + 2, last)
            k3 = jnp.minimum(k0 + 3, last)
            gather(0, k0).wait()
            store(0, k0).start()
            gather(1, k1).wait()
            store(1, k1).start()
            store(0, k0).wait()
            gather(0, k2).start()
            store(1, k1).wait()
            gather(1, k3).start()

        gather(0, b_per_w - 1).wait()
        gather(1, b_per_w - 1).wait()

    return run(table, idx2d)


def kernel(idx, logits_table):
    nb, s = idx.shape
    v, d = logits_table.shape
    del v
    b_per_w = nb // _NW
    table = jnp.pad(logits_table, ((0, 0), (0, _DPAD - d)))
    idx_p = jnp.pad(idx.astype(jnp.int32), ((0, 0), (0, _SPAD - s)), mode="edge")
    idx2d = idx_p.reshape(_NW, b_per_w * _SPAD)
    out = _gather_rows(table, idx2d, nb)
    return out[:, :s, :d]
